# Initial kernel scaffold; baseline (speedup 1.0000x reference)
import jax
import jax.numpy as jnp
from jax.experimental import pallas as pl


def kernel(x, W1, W2, W3, W4, W5, W6):
    raise NotImplementedError("write your pallas kernel here")


# speedup vs baseline: 1.0012x; 1.0012x over previous
import functools

import jax
import jax.numpy as jnp
from jax.experimental import pallas as pl

_K = 20


def _mish(x):
    sp = jnp.maximum(x, 0.0) + jnp.log1p(jnp.exp(-jnp.abs(x)))
    return x * jnp.tanh(sp)


def _knn_idx(x, k):
    inner = jnp.einsum('bdi,bdj->bij', x, 2.0 * x)
    xx = jnp.sum(jnp.square(x), axis=1)
    dist = xx[:, :, None] - inner + xx[:, None, :]
    _, idx = jax.lax.top_k(-dist, k + 1)
    return idx[:, :, 1:]


def _graph_feature(x, k):
    b, d, n = x.shape
    idx = _knn_idx(x, k)
    bi = jnp.arange(b)[:, None, None]
    f = x[bi, :, idx]
    f = jnp.transpose(f, (0, 3, 1, 2))
    xc = jnp.broadcast_to(x[:, :, :, None], (b, d, n, k))
    return jnp.concatenate([f - xc, xc], axis=1)


def _bn(x, axes):
    m = jnp.mean(x, axis=axes, keepdims=True)
    v = jnp.var(x, axis=axes, keepdims=True)
    return (x - m) / jnp.sqrt(v + 1e-5)


def _conv2d(x, W):
    return _mish(_bn(jnp.einsum('oc,bcnk->bonk', W, x), (0, 2, 3)))


def _moments_kernel(cat_ref, m_ref, s_ref):
    @pl.when(pl.program_id(0) == 0)
    def _init():
        m_ref[...] = jnp.zeros_like(m_ref)
        s_ref[...] = jnp.zeros_like(s_ref)

    c = cat_ref[0]
    m_ref[...] += jnp.dot(c, c.T, preferred_element_type=jnp.float32)
    s_ref[...] += jnp.sum(c, axis=1)[None, :]


def _fold_kernel(nvals_ref, m_ref, s_ref, w_ref, wp_ref, bp_ref):
    n_total = nvals_ref[0, 0]
    w = w_ref[...]
    mean_c = s_ref[0] / n_total
    mean_z = jnp.dot(w, mean_c[:, None],
                     preferred_element_type=jnp.float32)[:, 0]
    wm = jnp.dot(w, m_ref[...], preferred_element_type=jnp.float32)
    ez2 = jnp.sum(wm * w, axis=1) / n_total
    var = ez2 - mean_z * mean_z
    scale = jax.lax.rsqrt(var + 1e-5)
    wp_ref[...] = w * scale[:, None]
    bp_ref[...] = (-mean_z * scale)[None, :]


def _final_conv_kernel(cat_ref, wp_ref, bp_ref, out_ref):
    z = jnp.dot(wp_ref[...], cat_ref[0],
                preferred_element_type=jnp.float32) + bp_ref[0][:, None]
    out_ref[0] = _mish(z)


def _final_conv(cat, W6):
    b, c, n = cat.shape
    nb = 512
    grid = (b * (n // nb),)
    m, s = pl.pallas_call(
        _moments_kernel,
        grid=(b * (n // nb),),
        in_specs=[pl.BlockSpec((1, c, nb),
                               lambda i: (i // (n // nb), 0, i % (n // nb)))],
        out_specs=[pl.BlockSpec((c, c), lambda i: (0, 0)),
                   pl.BlockSpec((1, c), lambda i: (0, 0))],
        out_shape=[jax.ShapeDtypeStruct((c, c), jnp.float32),
                   jax.ShapeDtypeStruct((1, c), jnp.float32)],
    )(cat)
    nvals = jnp.full((1, 1), float(b * n), dtype=jnp.float32)
    wp, bp = pl.pallas_call(
        _fold_kernel,
        out_shape=[jax.ShapeDtypeStruct(W6.shape, jnp.float32),
                   jax.ShapeDtypeStruct((1, W6.shape[0]), jnp.float32)],
    )(nvals, m, s, W6)
    out = pl.pallas_call(
        _final_conv_kernel,
        grid=(b, n // nb),
        in_specs=[pl.BlockSpec((1, c, nb), lambda i, j: (i, 0, j)),
                  pl.BlockSpec(W6.shape, lambda i, j: (0, 0)),
                  pl.BlockSpec((1, W6.shape[0]), lambda i, j: (0, 0))],
        out_specs=pl.BlockSpec((1, W6.shape[0], nb), lambda i, j: (i, 0, j)),
        out_shape=jax.ShapeDtypeStruct((b, W6.shape[0], n), jnp.float32),
    )(cat, wp, bp)
    return out


def kernel(x, W1, W2, W3, W4, W5, W6):
    f = _graph_feature(x, _K)
    h = _conv2d(f, W1)
    h = _conv2d(h, W2)
    x1 = jnp.max(h, axis=-1)
    f = _graph_feature(x1, _K)
    h = _conv2d(f, W3)
    h = _conv2d(h, W4)
    x2 = jnp.max(h, axis=-1)
    f = _graph_feature(x2, _K)
    h = _conv2d(f, W5)
    x3 = jnp.max(h, axis=-1)
    cat = jnp.concatenate([x1, x2, x3], axis=1)
    return _final_conv(cat, W6)


# speedup vs baseline: 2.6929x; 2.6897x over previous
import functools

import jax
import jax.numpy as jnp
from jax.experimental import pallas as pl
from jax.experimental.pallas import tpu as pltpu

_K = 20


def _mish(x):
    sp = jnp.maximum(x, 0.0) + jnp.log1p(jnp.exp(-jnp.abs(x)))
    return x * jnp.tanh(sp)


def _knn_kernel(xq_ref, xa_ref, out_ref, work_ref, ids_ref, *, n, q, kp1):
    xq = xq_ref[0]
    xa = xa_ref[0]
    inner = jax.lax.dot_general(
        xq.astype(jnp.bfloat16), xa.astype(jnp.bfloat16),
        (((1,), (1,)), ((), ())),
        preferred_element_type=jnp.float32)
    qq = jnp.sum(xq * xq, axis=1)[:, None]
    aa = jnp.sum(xa * xa, axis=1)[None, :]
    work_ref[...] = qq - 2.0 * inner + aa
    lane = jax.lax.broadcasted_iota(jnp.int32, (q, n), 1)
    col = jax.lax.broadcasted_iota(jnp.int32, (q, ids_ref.shape[1]), 1)

    def body(j, _):
        w = work_ref[...]
        m = jnp.min(w, axis=1, keepdims=True)
        cand = jnp.where(w == m, lane, jnp.int32(n))
        i = jnp.min(cand, axis=1, keepdims=True)
        ids_ref[...] = jnp.where(col == j, i, ids_ref[...])
        work_ref[...] = jnp.where(lane == i, jnp.float32(jnp.inf), w)
        return 0

    jax.lax.fori_loop(0, kp1, body, 0, unroll=True)
    out_ref[0] = ids_ref[:, :kp1]


def _topk_idx(xr, kp1):
    b, n, d = xr.shape
    q = 256
    kern = functools.partial(_knn_kernel, n=n, q=q, kp1=kp1)
    return pl.pallas_call(
        kern,
        grid=(b, n // q),
        in_specs=[pl.BlockSpec((1, q, d), lambda i, j: (i, j, 0)),
                  pl.BlockSpec((1, n, d), lambda i, j: (i, 0, 0))],
        out_specs=pl.BlockSpec((1, q, kp1), lambda i, j: (i, j, 0)),
        out_shape=jax.ShapeDtypeStruct((b, n, kp1), jnp.int32),
        scratch_shapes=[pltpu.VMEM((q, n), jnp.float32),
                        pltpu.VMEM((q, 32), jnp.int32)],
    )(xr, xr)


def _knn_idx(x, k):
    xr = jnp.transpose(x, (0, 2, 1))
    idx = _topk_idx(xr, k + 1)
    return idx[:, :, 1:]


def _graph_feature(x, k):
    b, d, n = x.shape
    idx = _knn_idx(x, k)
    bi = jnp.arange(b)[:, None, None]
    f = x[bi, :, idx]
    f = jnp.transpose(f, (0, 3, 1, 2))
    xc = jnp.broadcast_to(x[:, :, :, None], (b, d, n, k))
    return jnp.concatenate([f - xc, xc], axis=1)


def _bn(x, axes):
    m = jnp.mean(x, axis=axes, keepdims=True)
    v = jnp.var(x, axis=axes, keepdims=True)
    return (x - m) / jnp.sqrt(v + 1e-5)


def _conv2d(x, W):
    return _mish(_bn(jnp.einsum('oc,bcnk->bonk', W, x), (0, 2, 3)))


def _moments_kernel(cat_ref, m_ref, s_ref):
    @pl.when(pl.program_id(0) == 0)
    def _init():
        m_ref[...] = jnp.zeros_like(m_ref)
        s_ref[...] = jnp.zeros_like(s_ref)

    c = cat_ref[0]
    m_ref[...] += jnp.dot(c, c.T, preferred_element_type=jnp.float32)
    s_ref[...] += jnp.sum(c, axis=1)[None, :]


def _fold_kernel(nvals_ref, m_ref, s_ref, w_ref, wp_ref, bp_ref):
    n_total = nvals_ref[0, 0]
    w = w_ref[...]
    mean_c = s_ref[0] / n_total
    mean_z = jnp.dot(w, mean_c[:, None],
                     preferred_element_type=jnp.float32)[:, 0]
    wm = jnp.dot(w, m_ref[...], preferred_element_type=jnp.float32)
    ez2 = jnp.sum(wm * w, axis=1) / n_total
    var = ez2 - mean_z * mean_z
    scale = jax.lax.rsqrt(var + 1e-5)
    wp_ref[...] = w * scale[:, None]
    bp_ref[...] = (-mean_z * scale)[None, :]


def _final_conv_kernel(cat_ref, wp_ref, bp_ref, out_ref):
    z = jnp.dot(wp_ref[...], cat_ref[0],
                preferred_element_type=jnp.float32) + bp_ref[0][:, None]
    out_ref[0] = _mish(z)


def _final_conv(cat, W6):
    b, c, n = cat.shape
    nb = 512
    grid = (b * (n // nb),)
    m, s = pl.pallas_call(
        _moments_kernel,
        grid=(b * (n // nb),),
        in_specs=[pl.BlockSpec((1, c, nb),
                               lambda i: (i // (n // nb), 0, i % (n // nb)))],
        out_specs=[pl.BlockSpec((c, c), lambda i: (0, 0)),
                   pl.BlockSpec((1, c), lambda i: (0, 0))],
        out_shape=[jax.ShapeDtypeStruct((c, c), jnp.float32),
                   jax.ShapeDtypeStruct((1, c), jnp.float32)],
    )(cat)
    nvals = jnp.full((1, 1), float(b * n), dtype=jnp.float32)
    wp, bp = pl.pallas_call(
        _fold_kernel,
        out_shape=[jax.ShapeDtypeStruct(W6.shape, jnp.float32),
                   jax.ShapeDtypeStruct((1, W6.shape[0]), jnp.float32)],
    )(nvals, m, s, W6)
    out = pl.pallas_call(
        _final_conv_kernel,
        grid=(b, n // nb),
        in_specs=[pl.BlockSpec((1, c, nb), lambda i, j: (i, 0, j)),
                  pl.BlockSpec(W6.shape, lambda i, j: (0, 0)),
                  pl.BlockSpec((1, W6.shape[0]), lambda i, j: (0, 0))],
        out_specs=pl.BlockSpec((1, W6.shape[0], nb), lambda i, j: (i, 0, j)),
        out_shape=jax.ShapeDtypeStruct((b, W6.shape[0], n), jnp.float32),
    )(cat, wp, bp)
    return out


def kernel(x, W1, W2, W3, W4, W5, W6):
    f = _graph_feature(x, _K)
    h = _conv2d(f, W1)
    h = _conv2d(h, W2)
    x1 = jnp.max(h, axis=-1)
    f = _graph_feature(x1, _K)
    h = _conv2d(f, W3)
    h = _conv2d(h, W4)
    x2 = jnp.max(h, axis=-1)
    f = _graph_feature(x2, _K)
    h = _conv2d(f, W5)
    x3 = jnp.max(h, axis=-1)
    cat = jnp.concatenate([x1, x2, x3], axis=1)
    return _final_conv(cat, W6)


# speedup vs baseline: 5.8028x; 2.1548x over previous
import functools

import jax
import jax.numpy as jnp
from jax.experimental import pallas as pl
from jax.experimental.pallas import tpu as pltpu
from jax.experimental.pallas import tpu_sc as plsc

_K = 20
_KP = 24
_B = 8
_N = 4096
_Q = 256
_P = 256
_NW = 32


def _mish(x):
    sp = jnp.maximum(x, 0.0) + jnp.log1p(jnp.exp(-jnp.abs(x)))
    return x * jnp.tanh(sp)


def _bf(x):
    return x.astype(jnp.bfloat16)


def _knn_kernel(xq_ref, xa_ref, xxq_ref, xxa_ref, out_ref, work_ref,
                ids_ref, *, n, q, kp1):
    xq = xq_ref[0]
    xa = xa_ref[0]
    inner = jax.lax.dot_general(
        _bf(xq), _bf(xa), (((1,), (1,)), ((), ())),
        preferred_element_type=jnp.float32)
    qq = xxq_ref[0][:, 0:1]
    aa = xxa_ref[0][0:1, :]
    work_ref[...] = qq - 2.0 * inner + aa
    lane = jax.lax.broadcasted_iota(jnp.int32, (q, n), 1)
    col = jax.lax.broadcasted_iota(jnp.int32, (q, ids_ref.shape[1]), 1)

    def body(j, _):
        w = work_ref[...]
        m = jnp.min(w, axis=1, keepdims=True)
        cand = jnp.where(w == m, lane, jnp.int32(n))
        i = jnp.min(cand, axis=1, keepdims=True)
        ids_ref[...] = jnp.where(col == j, i, ids_ref[...])
        work_ref[...] = jnp.where(lane == i, jnp.float32(jnp.inf), w)
        return 0

    jax.lax.fori_loop(0, kp1, body, 0, unroll=True)
    base = pl.program_id(0) * n
    colk = jax.lax.broadcasted_iota(jnp.int32, (q, _KP), 1)
    shifted = ids_ref[:, 1:1 + _KP]
    out_ref[0] = jnp.where(colk < kp1 - 1, shifted + base, 0)


def _topk_gids(xr, xx, kp1):
    b, n, d = xr.shape
    q = _Q
    kern = functools.partial(_knn_kernel, n=n, q=q, kp1=kp1)
    xxq = xx.reshape(b, n, 1)
    xxa = xx.reshape(b, 1, n)
    return pl.pallas_call(
        kern,
        grid=(b, n // q),
        in_specs=[pl.BlockSpec((1, q, d), lambda i, j: (i, j, 0)),
                  pl.BlockSpec((1, n, d), lambda i, j: (i, 0, 0)),
                  pl.BlockSpec((1, q, 1), lambda i, j: (i, j, 0)),
                  pl.BlockSpec((1, 1, n), lambda i, j: (i, 0, 0))],
        out_specs=pl.BlockSpec((1, q, _KP), lambda i, j: (i, j, 0)),
        out_shape=jax.ShapeDtypeStruct((b, n, _KP), jnp.int32),
        scratch_shapes=[pltpu.VMEM((q, n), jnp.float32),
                        pltpu.VMEM((q, 32), jnp.int32)],
    )(xr, xr, xxq, xxa)


def _sc_gather(table, gid, dp):
    rows = gid.shape[0]
    per_w = rows // _NW
    ch = 128
    iters = per_w // ch
    mesh = plsc.VectorSubcoreMesh(core_axis_name="c", subcore_axis_name="s")

    @functools.partial(
        pl.kernel, mesh=mesh,
        out_type=jax.ShapeDtypeStruct((rows, dp), jnp.float32),
        scratch_types=[pltpu.VMEM((ch,), jnp.int32),
                       pltpu.VMEM((ch, dp), jnp.float32),
                       pltpu.SemaphoreType.DMA],
        compiler_params=pltpu.CompilerParams(use_tc_tiling_on_sc=False),
    )
    def k(table_hbm, gid_hbm, out_hbm, idx_v, rows_v, sem):
        wid = jax.lax.axis_index("s") * 2 + jax.lax.axis_index("c")
        w0 = wid * per_w

        def body(it, _):
            base = w0 + it * ch
            pltpu.sync_copy(gid_hbm.at[pl.ds(base, ch)], idx_v)
            pltpu.async_copy(table_hbm.at[idx_v], rows_v, sem).wait()
            pltpu.sync_copy(rows_v, out_hbm.at[pl.ds(base, ch)])
            return 0

        jax.lax.fori_loop(0, iters, body, 0)

    return k(table, gid)


def _z1_block(zg_ref, xr_ref, wt_ref, p, dp, d):
    xc = xr_ref[...]
    zg = zg_ref[...].reshape(p, _KP, dp)
    xcr = jnp.broadcast_to(xc[:, None, :], (p, _KP, dp))
    fsub = (zg - xcr).reshape(p * _KP, dp)
    f = jnp.concatenate(
        [fsub[:, :d], xcr.reshape(p * _KP, dp)[:, :d]], axis=1)
    z1 = jnp.dot(_bf(f), _bf(wt_ref[...]),
                 preferred_element_type=jnp.float32)
    return z1.reshape(p, _KP, 64)


def _bn_apply(z, st_ref, nval):
    m = st_ref[0:1, :]
    var = st_ref[1:2, :]
    return (z - m[None]) / jnp.sqrt(var + 1e-5)[None]


def _masked(z, p, fill):
    kio = jax.lax.broadcasted_iota(jnp.int32, (p, _KP, 1), 1)
    return jnp.where(kio < _K, z, fill)


def _block_sums(z, p):
    zf = z.reshape(p * _KP, 64)
    both = jnp.concatenate([zf, zf * zf], axis=1)
    ones = jnp.ones((1, p * _KP), jnp.float32)
    s = jax.lax.dot_general(ones, both, (((1,), (0,)), ((), ())),
                            precision=jax.lax.Precision.HIGHEST,
                            preferred_element_type=jnp.float32)
    return s.reshape(1, 2, 64)


def _tree_reduce_kernel(ps_ref, st_ref):
    a = ps_ref[...]
    m = a.shape[0]
    while m > 1:
        h = m // 2
        a = a[:h] + a[h:m]
        m = h
    st_ref[...] = a[0]


def _zwrite1_kernel(zg_ref, xr_ref, wt_ref, z_ref, *, p, dp, d):
    z1 = _z1_block(zg_ref, xr_ref, wt_ref, p, dp, d)
    z_ref[...] = z1.reshape(p * _KP, 64)


def _zwrite2_kernel(zg_ref, xr_ref, wt_ref, st1_ref, w2t_ref,
                    z_ref, *, p, dp, d, nval):
    z1 = _z1_block(zg_ref, xr_ref, wt_ref, p, dp, d)
    h1 = _mish(_bn_apply(z1, st1_ref, nval))
    z2 = jnp.dot(_bf(h1.reshape(p * _KP, 64)), _bf(w2t_ref[...]),
                 preferred_element_type=jnp.float32)
    z_ref[...] = z2


def _max2_kernel(zg_ref, xr_ref, wt_ref, st1_ref, w2t_ref,
                 st2_ref, out_ref, *, p, dp, d, nval):
    z1 = _z1_block(zg_ref, xr_ref, wt_ref, p, dp, d)
    h1 = _mish(_bn_apply(z1, st1_ref, nval))
    z2 = jnp.dot(_bf(h1.reshape(p * _KP, 64)), _bf(w2t_ref[...]),
                 preferred_element_type=jnp.float32).reshape(p, _KP, 64)
    h2 = _mish(_bn_apply(z2, st2_ref, nval))
    out_ref[...] = jnp.max(_masked(h2, p, -jnp.inf), axis=1)


def _max1_kernel(zg_ref, xr_ref, wt_ref, st1_ref, out_ref,
                 *, p, dp, d, nval):
    z1 = _z1_block(zg_ref, xr_ref, wt_ref, p, dp, d)
    h1 = _mish(_bn_apply(z1, st1_ref, nval))
    out_ref[...] = jnp.max(_masked(h1, p, -jnp.inf), axis=1)


def _stage(xr, wt, w2t, d, xx):
    bn, dp = xr.shape
    p = _P
    grid = (bn // p,)
    nval = float(bn * _K)
    gids = _topk_gids(xr.reshape(_B, _N, dp), xx, _K + 1)
    zg = _sc_gather(xr, gids.reshape(-1), dp)

    zspec = pl.BlockSpec((p * _KP, dp), lambda i: (i, 0))
    xspec = pl.BlockSpec((p, dp), lambda i: (i, 0))
    w_spec = pl.BlockSpec(wt.shape, lambda i: (0, 0))
    st_spec = pl.BlockSpec((2, 64), lambda i: (0, 0))
    zout_spec = pl.BlockSpec((p * _KP, 64), lambda i: (i, 0))
    zout_shape = jax.ShapeDtypeStruct((bn * _KP, 64), jnp.float32)

    def _xla_stats(zarr):
        zt = jnp.transpose(
            zarr.reshape(_B, _N, _KP, 64)[:, :, :_K, :], (0, 3, 1, 2))
        m = jnp.mean(zt, axis=(0, 2, 3))
        v = jnp.var(zt, axis=(0, 2, 3))
        return jnp.stack([m, v])

    z1arr = pl.pallas_call(
        functools.partial(_zwrite1_kernel, p=p, dp=dp, d=d),
        grid=grid,
        in_specs=[zspec, xspec, w_spec],
        out_specs=zout_spec, out_shape=zout_shape,
    )(zg, xr, wt)
    st1 = _xla_stats(z1arr)

    if w2t is None:
        return pl.pallas_call(
            functools.partial(_max1_kernel, p=p, dp=dp, d=d, nval=nval),
            grid=grid,
            in_specs=[zspec, xspec, w_spec, st_spec],
            out_specs=pl.BlockSpec((p, 64), lambda i: (i, 0)),
            out_shape=jax.ShapeDtypeStruct((bn, 64), jnp.float32),
        )(zg, xr, wt, st1)

    w2_spec = pl.BlockSpec((64, 64), lambda i: (0, 0))
    z2arr = pl.pallas_call(
        functools.partial(_zwrite2_kernel, p=p, dp=dp, d=d, nval=nval),
        grid=grid,
        in_specs=[zspec, xspec, w_spec, st_spec, w2_spec],
        out_specs=zout_spec, out_shape=zout_shape,
    )(zg, xr, wt, st1, w2t)
    st2 = _xla_stats(z2arr)

    return pl.pallas_call(
        functools.partial(_max2_kernel, p=p, dp=dp, d=d, nval=nval),
        grid=grid,
        in_specs=[zspec, xspec, w_spec, st_spec, w2_spec, st_spec],
        out_specs=pl.BlockSpec((p, 64), lambda i: (i, 0)),
        out_shape=jax.ShapeDtypeStruct((bn, 64), jnp.float32),
    )(zg, xr, wt, st1, w2t, st2)


def _moments_kernel(cat_ref, m_ref, s_ref):
    @pl.when(pl.program_id(0) == 0)
    def _init():
        m_ref[...] = jnp.zeros_like(m_ref)
        s_ref[...] = jnp.zeros_like(s_ref)

    c = cat_ref[...]
    cb = _bf(c)
    m_ref[...] += jax.lax.dot_general(
        cb, cb, (((0,), (0,)), ((), ())),
        preferred_element_type=jnp.float32)
    s_ref[...] += jnp.sum(c, axis=0)[None, :]


def _fold_kernel(m_ref, s_ref, w_ref, msc_ref, *, nval):
    w = w_ref[...]
    mean_c = s_ref[...] / nval
    mean_z = jax.lax.dot_general(
        w, mean_c, (((1,), (1,)), ((), ())),
        preferred_element_type=jnp.float32)
    wm = jax.lax.dot_general(
        w, m_ref[...], (((1,), (0,)), ((), ())),
        preferred_element_type=jnp.float32)
    ez2 = jnp.sum(wm * w, axis=1, keepdims=True) / nval
    var = ez2 - mean_z * mean_z
    r = 1.0 / jnp.sqrt(var + 1e-5)
    msc_ref[...] = jnp.concatenate([mean_z, r], axis=1)


def _final_conv_kernel(cat_ref, w_ref, msc_ref, out_ref):
    z = jax.lax.dot_general(
        _bf(w_ref[...]), _bf(cat_ref[...]), (((1,), (1,)), ((), ())),
        preferred_element_type=jnp.float32)
    m = msc_ref[:, 0:1]
    r = msc_ref[:, 1:2]
    out_ref[0] = _mish((z - m) * r)


def _final_conv(cat, W6):
    bn, c = cat.shape
    o = W6.shape[0]
    rp = min(2048, bn)
    m, s = pl.pallas_call(
        _moments_kernel,
        grid=(bn // rp,),
        in_specs=[pl.BlockSpec((rp, c), lambda i: (i, 0))],
        out_specs=[pl.BlockSpec((c, c), lambda i: (0, 0)),
                   pl.BlockSpec((1, c), lambda i: (0, 0))],
        out_shape=[jax.ShapeDtypeStruct((c, c), jnp.float32),
                   jax.ShapeDtypeStruct((1, c), jnp.float32)],
    )(cat)
    msc = pl.pallas_call(
        functools.partial(_fold_kernel, nval=float(bn)),
        out_shape=jax.ShapeDtypeStruct((o, 2), jnp.float32),
    )(m, s, W6)
    p4 = 512
    nb4 = _N // p4
    return pl.pallas_call(
        _final_conv_kernel,
        grid=(_B * nb4,),
        in_specs=[pl.BlockSpec((p4, c), lambda i: (i, 0)),
                  pl.BlockSpec((o, c), lambda i: (0, 0)),
                  pl.BlockSpec((o, 2), lambda i: (0, 0))],
        out_specs=pl.BlockSpec((1, o, p4), lambda i: (i // nb4, 0, i % nb4)),
        out_shape=jax.ShapeDtypeStruct((_B, o, _N), jnp.float32),
    )(cat, W6, msc)


def kernel(x, W1, W2, W3, W4, W5, W6):
    b, c, n = x.shape
    xr1 = jnp.transpose(x, (0, 2, 1)).reshape(b * n, c)
    xr1 = jnp.pad(xr1, ((0, 0), (0, 16 - c)))

    xx1 = jnp.sum(jnp.square(x), axis=1)
    x1 = _stage(xr1, W1.T, W2.T, c, xx1)
    x1_bdn = jnp.transpose(x1.reshape(b, n, 64), (0, 2, 1))
    xx2 = jnp.sum(jnp.square(x1_bdn), axis=1)
    x2 = _stage(x1, W3.T, W4.T, 64, xx2)
    x2_bdn = jnp.transpose(x2.reshape(b, n, 64), (0, 2, 1))
    xx3 = jnp.sum(jnp.square(x2_bdn), axis=1)
    x3 = _stage(x2, W5.T, None, 64, xx3)
    cat = jnp.concatenate([x1, x2, x3], axis=1)
    return _final_conv(cat, W6)
